# Initial kernel scaffold; baseline (speedup 1.0000x reference)
import jax
import jax.numpy as jnp
from jax.experimental import pallas as pl


def kernel(x, params, conns):
    raise NotImplementedError("write your pallas kernel here")


# speedup vs baseline: 10.3666x; 10.3666x over previous
import functools

import jax
import jax.numpy as jnp
from jax import lax
from jax.experimental import pallas as pl
from jax.experimental.pallas import tpu as pltpu
from jax.experimental.pallas import tpu_sc as plsc

_B = 512
_NG = 3008
_DD = 2048
_K = 6
_T1, _T2 = 1000, 250
_F1, _F2 = 16, 8
_T1P = 1024
_T2P = 256
_NW = 32


def _sc_gather(table, idx2, n_pad, d, chunk, nchunks):
  per_w = nchunks * chunk
  mesh = plsc.VectorSubcoreMesh(core_axis_name="c", subcore_axis_name="s")

  @functools.partial(
      pl.kernel,
      out_type=jax.ShapeDtypeStruct((n_pad, d), jnp.float32),
      mesh=mesh,
      scratch_types=[
          pltpu.VMEM((nchunks, chunk), jnp.int32),
          pltpu.VMEM((chunk, d), jnp.float32),
          pltpu.VMEM((chunk, d), jnp.float32),
          pltpu.SemaphoreType.DMA,
          pltpu.SemaphoreType.DMA,
      ],
  )
  def gather_k(table_hbm, idx_hbm, out_hbm, idx_v, buf0, buf1, sem0, sem1):
    wid = lax.axis_index("s") * 2 + lax.axis_index("c")
    rowbase = wid * per_w
    pltpu.sync_copy(idx_hbm.at[pl.ds(wid * nchunks, nchunks)], idx_v)
    bufs = (buf0, buf1)
    sems = (sem0, sem1)
    cps = [None, None]
    cps[0] = pltpu.async_copy(table_hbm.at[idx_v.at[0]], buf0, sem0)
    for c in range(nchunks):
      cur = c % 2
      nxt = (c + 1) % 2
      if c + 1 < nchunks:
        cps[nxt] = pltpu.async_copy(
            table_hbm.at[idx_v.at[c + 1]], bufs[nxt], sems[nxt])
      cps[cur].wait()
      pltpu.sync_copy(bufs[cur], out_hbm.at[pl.ds(rowbase + c * chunk, chunk)])

  return gather_k(table, idx2)


def _bn_lanes(z, ga, be):
  m = jnp.mean(z, axis=-1, keepdims=True)
  dd = z - m
  v = jnp.mean(dd * dd, axis=-1, keepdims=True)
  return ga * dd * lax.rsqrt(v + 1e-5) + be


def _l1_body(g_ref, w_ref, b_ref, ga_ref, be_ref, o_ref):
  tb = w_ref.shape[0]
  g = g_ref[...].reshape(tb, _F1, _B)
  w = w_ref[...].reshape(tb, _F1, _K)
  acc = b_ref[...][:, :, None] * jnp.ones((tb, _K, _B), jnp.float32)
  for i in range(_F1):
    acc = acc + w[:, i, :, None] * g[:, i, None, :]
  z = jnp.tanh(acc)
  h = _bn_lanes(z, ga_ref[...][:, :, None], be_ref[...][:, :, None])
  o_ref[...] = h.reshape(tb * _K, _B)


def _l2_body(g_ref, w_ref, b_ref, ga_ref, be_ref, o_ref):
  ob = w_ref.shape[0]
  g = g_ref[...].reshape(ob, _F2, _K * _B)
  w = w_ref[...].reshape(ob, _F2, _K, _K)
  acc = b_ref[...][:, :, None] * jnp.ones((ob, _K, _B), jnp.float32)
  for j in range(_F2):
    for q in range(_K):
      acc = acc + (w[:, j, :, q][:, :, None]
                   * g[:, j, q * _B:(q + 1) * _B][:, None, :])
  z = jnp.tanh(acc)
  h = _bn_lanes(z, ga_ref[...][:, :, None], be_ref[...][:, :, None])
  o_ref[...] = h.reshape(ob * _K, _B)


def _head_body(h2_ref, dr_ref, w3_ref, b3_ref, g3_ref, e3_ref,
               wd1_ref, bd1_ref, gd1_ref, ed1_ref,
               wd2_ref, bd2_ref, gd2_ref, ed2_ref,
               wd3_ref, bd3_ref, gd3_ref, ed3_ref,
               wf_ref, bf_ref, gf_ref, ef_ref,
               wa_ref, ba_ref, wo_ref, bo_ref, o_ref):
  def dot(a, b):
    return lax.dot(a, b, precision=lax.Precision.HIGHEST,
                   preferred_element_type=jnp.float32)

  h3 = _bn_lanes(jnp.tanh(dot(w3_ref[...], h2_ref[...]) + b3_ref[...]),
                 g3_ref[...], e3_ref[...])
  d = dr_ref[...]
  d = _bn_lanes(jnp.tanh(dot(wd1_ref[...], d) + bd1_ref[...]),
                gd1_ref[...], ed1_ref[...])
  d = _bn_lanes(jnp.tanh(dot(wd2_ref[...], d) + bd2_ref[...]),
                gd2_ref[...], ed2_ref[...])
  d = _bn_lanes(jnp.tanh(dot(wd3_ref[...], d) + bd3_ref[...]),
                gd3_ref[...], ed3_ref[...])
  f = jnp.concatenate([h3, d], axis=0)
  zf = _bn_lanes(jnp.tanh(dot(wf_ref[...], f) + bf_ref[...]),
                 gf_ref[...], ef_ref[...])
  oa = jnp.tanh(dot(wa_ref[...], zf) + ba_ref[...])
  o_ref[...] = wo_ref[0, 0] * oa + bo_ref[0, 0]


def _full_spec(shape):
  return pl.BlockSpec(shape, lambda: tuple(0 for _ in shape))


def kernel(x, params, conns):
  p = params

  xt = x.T
  genet = xt[:_NG]
  drugt = xt[_NG:_NG + _DD]

  sel1 = conns["cols1"][::_K]
  sel2 = conns["cols2"][::_K * _K] // _K

  idx1 = jnp.zeros((_T1P * _F1,), jnp.int32).at[:_T1 * _F1].set(sel1)
  g1 = _sc_gather(genet, idx1.reshape(-1, 64), _T1P * _F1, _B, 64, 8)

  w1m = jnp.zeros((_T1P, _F1 * _K), jnp.float32).at[:_T1].set(
      p["w1"].reshape(_T1, _F1 * _K))
  b1m = jnp.zeros((_T1P, _K), jnp.float32).at[:_T1].set(p["b1"].reshape(_T1, _K))
  ga1m = jnp.zeros((_T1P, _K), jnp.float32).at[:_T1].set(p["g1"].reshape(_T1, _K))
  be1m = jnp.zeros((_T1P, _K), jnp.float32).at[:_T1].set(p["be1"].reshape(_T1, _K))

  tb1 = 32
  n1 = _T1P // tb1
  h1 = pl.pallas_call(
      _l1_body,
      grid=(n1,),
      in_specs=[
          pl.BlockSpec((tb1 * _F1, _B), lambda i: (i, 0)),
          pl.BlockSpec((tb1, _F1 * _K), lambda i: (i, 0)),
          pl.BlockSpec((tb1, _K), lambda i: (i, 0)),
          pl.BlockSpec((tb1, _K), lambda i: (i, 0)),
          pl.BlockSpec((tb1, _K), lambda i: (i, 0)),
      ],
      out_specs=pl.BlockSpec((tb1 * _K, _B), lambda i: (i, 0)),
      out_shape=jax.ShapeDtypeStruct((_T1P * _K, _B), jnp.float32),
  )(g1, w1m, b1m, ga1m, be1m)

  h1tab = h1.reshape(_T1P, _K * _B)
  idx2 = jnp.zeros((_T2P * _F2,), jnp.int32).at[:_T2 * _F2].set(sel2)
  g2 = _sc_gather(h1tab, idx2.reshape(-1, 16), _T2P * _F2, _K * _B, 16, 4)

  w2m = jnp.zeros((_T2P, _F2 * _K * _K), jnp.float32).at[:_T2].set(
      p["w2"].reshape(_T2, _F2 * _K * _K))
  b2m = jnp.zeros((_T2P, _K), jnp.float32).at[:_T2].set(p["b2"].reshape(_T2, _K))
  ga2m = jnp.zeros((_T2P, _K), jnp.float32).at[:_T2].set(p["g2"].reshape(_T2, _K))
  be2m = jnp.zeros((_T2P, _K), jnp.float32).at[:_T2].set(p["be2"].reshape(_T2, _K))

  ob2 = 32
  n2 = _T2P // ob2
  h2 = pl.pallas_call(
      _l2_body,
      grid=(n2,),
      in_specs=[
          pl.BlockSpec((ob2 * _F2, _K * _B), lambda i: (i, 0)),
          pl.BlockSpec((ob2, _F2 * _K * _K), lambda i: (i, 0)),
          pl.BlockSpec((ob2, _K), lambda i: (i, 0)),
          pl.BlockSpec((ob2, _K), lambda i: (i, 0)),
          pl.BlockSpec((ob2, _K), lambda i: (i, 0)),
      ],
      out_specs=pl.BlockSpec((ob2 * _K, _B), lambda i: (i, 0)),
      out_shape=jax.ShapeDtypeStruct((_T2P * _K, _B), jnp.float32),
  )(g2, w2m, b2m, ga2m, be2m)

  w3t = jnp.zeros((_K, _T2P * _K), jnp.float32).at[:, :_T2 * _K].set(
      p["w3"].reshape(_T2, _K, _K).transpose(1, 0, 2).reshape(_K, _T2 * _K))
  args = [
      h2, drugt,
      w3t, p["b3"][:, None], p["g3"][:, None], p["be3"][:, None],
      p["Wd1"].T, p["bd1"][:, None], p["gd1"][:, None], p["bed1"][:, None],
      p["Wd2"].T, p["bd2"][:, None], p["gd2"][:, None], p["bed2"][:, None],
      p["Wd3"].T, p["bd3"][:, None], p["gd3"][:, None], p["bed3"][:, None],
      p["Wf"].T, p["bf"][:, None], p["gf"][:, None], p["bef"][:, None],
      p["Wa"].T, p["ba"][:, None], p["Wo"], p["bo"][:, None],
  ]
  out = pl.pallas_call(
      _head_body,
      in_specs=[_full_spec(tuple(a.shape)) for a in args],
      out_specs=_full_spec((1, _B)),
      out_shape=jax.ShapeDtypeStruct((1, _B), jnp.float32),
  )(*args)
  return out.reshape(_B, 1)
